# baseline (speedup 1.0000x reference)
import jax
import jax.numpy as jnp
import numpy as np
from jax.experimental import pallas as pl
from jax.experimental.pallas import tpu as pltpu

_N = 2048
_IN = 512
_T = 32
_ALPHA = 0.9
_BETA = 0.9
_TH = 20.0
_DELTA = float((np.exp(-1.0 / 20.0) - np.exp(1.0 / 20.0)) * 0.001)

_WBLK = 256
_TCHUNK = 128


def _scan_kernel(x_ref, win_ref, b1_ref, v_ref, brec_ref, spk_ref,
                 curr_ref, vhiT_ref, vloT_ref):
    for c in range(_N // _TCHUNK):
        blk = v_ref[pl.ds(c * _TCHUNK, _TCHUNK), :]
        blkT = blk.T
        hi = blkT.astype(jnp.bfloat16)
        lo = (blkT - hi.astype(jnp.float32)).astype(jnp.bfloat16)
        vhiT_ref[:, pl.ds(c * _TCHUNK, _TCHUNK)] = hi
        vloT_ref[:, pl.ds(c * _TCHUNK, _TCHUNK)] = lo

    curr_ref[:] = (
        jax.lax.dot_general(
            x_ref[:], win_ref[:], (((1,), (1,)), ((), ())),
            preferred_element_type=jnp.float32,
        )
        + b1_ref[:]
    )

    z = jnp.zeros((1, _N), jnp.float32)

    def body(t, carry):
        spk, syn, mem = carry
        s16 = spk.astype(jnp.bfloat16)
        rec = (
            jax.lax.dot_general(
                s16, vhiT_ref[:], (((1,), (0,)), ((), ())),
                preferred_element_type=jnp.float32,
            )
            + jax.lax.dot_general(
                s16, vloT_ref[:], (((1,), (0,)), ((), ())),
                preferred_element_type=jnp.float32,
            )
            + brec_ref[:]
        )
        syn = _ALPHA * syn + curr_ref[pl.ds(t, 1), :] + rec
        mem = _BETA * mem + syn - spk * _TH
        spk = ((mem - _TH) > 0.0).astype(jnp.float32)
        spk_ref[pl.ds(t, 1), :] = spk
        return (spk, syn, mem)

    jax.lax.fori_loop(0, _T, body, (z, z, z), unroll=False)


def _stdp_kernel(sT_ref, s_ref, w_ref, out_ref):
    st1 = sT_ref[:, 1:_T]
    s0 = s_ref[0 : _T - 1, :]
    c = jax.lax.dot_general(
        st1, s0, (((1,), (0,)), ((), ())), preferred_element_type=jnp.float32
    )
    out_ref[:] = jnp.clip(w_ref[:] + _DELTA * c, -1.0, 1.0)


def kernel(x, Win, b1, V, b_rec, Wlsm):
    x2 = x.reshape(_T, _IN)
    b1r = b1.reshape(1, _N)
    brr = b_rec.reshape(1, _N)

    spk_rec = pl.pallas_call(
        _scan_kernel,
        out_shape=jax.ShapeDtypeStruct((_T, _N), jnp.float32),
        scratch_shapes=[
            pltpu.VMEM((_T, _N), jnp.float32),
            pltpu.VMEM((_N, _N), jnp.bfloat16),
            pltpu.VMEM((_N, _N), jnp.bfloat16),
        ],
    )(x2, Win, b1r, V, brr)

    spk_recT = spk_rec.T

    nblk = _N // _WBLK
    w_final = pl.pallas_call(
        _stdp_kernel,
        grid=(nblk,),
        in_specs=[
            pl.BlockSpec((_WBLK, _T), lambda i: (i, 0)),
            pl.BlockSpec((_T, _N), lambda i: (0, 0)),
            pl.BlockSpec((_WBLK, _N), lambda i: (i, 0)),
        ],
        out_specs=pl.BlockSpec((_WBLK, _N), lambda i: (i, 0)),
        out_shape=jax.ShapeDtypeStruct((_N, _N), jnp.float32),
    )(spk_recT, spk_rec, Wlsm)

    return spk_rec.reshape(_T, 1, _N), w_final


# speedup vs baseline: 1.5712x; 1.5712x over previous
import jax
import jax.numpy as jnp
import numpy as np
from jax.experimental import pallas as pl
from jax.experimental.pallas import tpu as pltpu

_N = 2048
_IN = 512
_T = 32
_ALPHA = 0.9
_BETA = 0.9
_TH = 20.0
_DELTA = float((np.exp(-1.0 / 20.0) - np.exp(1.0 / 20.0)) * 0.001)

_WBLK = 256
_TCHUNK = 128


def _scan_kernel(x_ref, win_ref, b1_ref, v_ref, brec_ref, spk_ref,
                 curr_ref, vT_ref):
    for c in range(_N // _TCHUNK):
        vT_ref[:, pl.ds(c * _TCHUNK, _TCHUNK)] = (
            v_ref[pl.ds(c * _TCHUNK, _TCHUNK), :].T
        )

    curr_ref[:] = (
        jax.lax.dot_general(
            x_ref[:], win_ref[:], (((1,), (1,)), ((), ())),
            preferred_element_type=jnp.float32,
        )
        + b1_ref[:]
    )

    z = jnp.zeros((1, _N), jnp.float32)

    def body(t, carry):
        spk, syn, mem = carry

        def live(_):
            return jax.lax.dot_general(
                spk, vT_ref[:], (((1,), (0,)), ((), ())),
                preferred_element_type=jnp.float32,
            )

        def silent(_):
            return jnp.zeros((1, _N), jnp.float32)

        any_spike = jnp.sum(spk) > 0.0
        rec = jax.lax.cond(any_spike, live, silent, None) + brec_ref[:]
        syn = _ALPHA * syn + curr_ref[pl.ds(t, 1), :] + rec
        mem = _BETA * mem + syn - spk * _TH
        spk = ((mem - _TH) > 0.0).astype(jnp.float32)
        spk_ref[pl.ds(t, 1), :] = spk
        return (spk, syn, mem)

    jax.lax.fori_loop(0, _T, body, (z, z, z), unroll=False)


def _stdp_kernel(s_ref, w_ref, out_ref):
    i = pl.program_id(0)
    s1 = s_ref[1:_T, pl.ds(i * _WBLK, _WBLK)]
    st1 = s1.T
    s0 = s_ref[0 : _T - 1, :]
    c = jax.lax.dot_general(
        st1, s0, (((1,), (0,)), ((), ())), preferred_element_type=jnp.float32
    )
    out_ref[:] = jnp.clip(w_ref[:] + _DELTA * c, -1.0, 1.0)


def kernel(x, Win, b1, V, b_rec, Wlsm):
    x2 = x.reshape(_T, _IN)
    b1r = b1.reshape(1, _N)
    brr = b_rec.reshape(1, _N)

    spk_rec = pl.pallas_call(
        _scan_kernel,
        out_shape=jax.ShapeDtypeStruct((_T, _N), jnp.float32),
        scratch_shapes=[
            pltpu.VMEM((_T, _N), jnp.float32),
            pltpu.VMEM((_N, _N), jnp.float32),
        ],
    )(x2, Win, b1r, V, brr)

    nblk = _N // _WBLK
    w_final = pl.pallas_call(
        _stdp_kernel,
        grid=(nblk,),
        in_specs=[
            pl.BlockSpec((_T, _N), lambda i: (0, 0)),
            pl.BlockSpec((_WBLK, _N), lambda i: (i, 0)),
        ],
        out_specs=pl.BlockSpec((_WBLK, _N), lambda i: (i, 0)),
        out_shape=jax.ShapeDtypeStruct((_N, _N), jnp.float32),
    )(spk_rec, Wlsm)

    return spk_rec.reshape(_T, 1, _N), w_final


# speedup vs baseline: 1.6097x; 1.0245x over previous
import jax
import jax.numpy as jnp
import numpy as np
from jax.experimental import pallas as pl
from jax.experimental.pallas import tpu as pltpu

_N = 2048
_IN = 512
_T = 32
_ALPHA = 0.9
_BETA = 0.9
_TH = 20.0
_DELTA = float((np.exp(-1.0 / 20.0) - np.exp(1.0 / 20.0)) * 0.001)

_WBLK = 256
_TCHUNK = 128


def _fused_kernel(x_ref, win_ref, b1_ref, v_ref, brec_ref, w_ref,
                  spk_ref, wout_ref, curr_ref, vT_ref):
    i = pl.program_id(0)

    @pl.when(i == 0)
    def _scan():
        for c in range(_N // _TCHUNK):
            vT_ref[:, pl.ds(c * _TCHUNK, _TCHUNK)] = (
                v_ref[pl.ds(c * _TCHUNK, _TCHUNK), :].T
            )

        curr_ref[:] = (
            jax.lax.dot_general(
                x_ref[:], win_ref[:], (((1,), (1,)), ((), ())),
                preferred_element_type=jnp.float32,
            )
            + b1_ref[:]
        )

        z = jnp.zeros((1, _N), jnp.float32)

        def body(t, carry):
            spk, syn, mem = carry

            def live(_):
                return jax.lax.dot_general(
                    spk, vT_ref[:], (((1,), (0,)), ((), ())),
                    preferred_element_type=jnp.float32,
                )

            def silent(_):
                return jnp.zeros((1, _N), jnp.float32)

            any_spike = jnp.sum(spk) > 0.0
            rec = jax.lax.cond(any_spike, live, silent, None) + brec_ref[:]
            syn = _ALPHA * syn + curr_ref[pl.ds(t, 1), :] + rec
            mem = _BETA * mem + syn - spk * _TH
            spk = ((mem - _TH) > 0.0).astype(jnp.float32)
            spk_ref[pl.ds(t, 1), :] = spk
            return (spk, syn, mem)

        jax.lax.fori_loop(0, _T, body, (z, z, z), unroll=False)

    s1 = spk_ref[1:_T, pl.ds(i * _WBLK, _WBLK)]
    st1 = s1.T
    s0 = spk_ref[0 : _T - 1, :]
    c = jax.lax.dot_general(
        st1, s0, (((1,), (0,)), ((), ())), preferred_element_type=jnp.float32
    )
    wout_ref[:] = jnp.clip(w_ref[:] + _DELTA * c, -1.0, 1.0)


def kernel(x, Win, b1, V, b_rec, Wlsm):
    x2 = x.reshape(_T, _IN)
    b1r = b1.reshape(1, _N)
    brr = b_rec.reshape(1, _N)

    nblk = _N // _WBLK
    spk_rec, w_final = pl.pallas_call(
        _fused_kernel,
        grid=(nblk,),
        in_specs=[
            pl.BlockSpec((_T, _IN), lambda i: (0, 0)),
            pl.BlockSpec((_N, _IN), lambda i: (0, 0)),
            pl.BlockSpec((1, _N), lambda i: (0, 0)),
            pl.BlockSpec((_N, _N), lambda i: (0, 0)),
            pl.BlockSpec((1, _N), lambda i: (0, 0)),
            pl.BlockSpec((_WBLK, _N), lambda i: (i, 0)),
        ],
        out_specs=[
            pl.BlockSpec((_T, _N), lambda i: (0, 0)),
            pl.BlockSpec((_WBLK, _N), lambda i: (i, 0)),
        ],
        out_shape=[
            jax.ShapeDtypeStruct((_T, _N), jnp.float32),
            jax.ShapeDtypeStruct((_N, _N), jnp.float32),
        ],
        scratch_shapes=[
            pltpu.VMEM((_T, _N), jnp.float32),
            pltpu.VMEM((_N, _N), jnp.float32),
        ],
    )(x2, Win, b1r, V, brr, Wlsm)

    return spk_rec.reshape(_T, 1, _N), w_final
